# Initial kernel scaffold; baseline (speedup 1.0000x reference)
import jax
import jax.numpy as jnp
from jax.experimental import pallas as pl


def kernel(x, fc_weight, bias):
    raise NotImplementedError("write your pallas kernel here")


# speedup vs baseline: 2.3073x; 2.3073x over previous
import functools

import jax
import jax.numpy as jnp
from jax import lax
from jax.experimental import pallas as pl
from jax.experimental.pallas import tpu as pltpu
from jax.experimental.pallas import tpu_sc as plsc

B = 16384
F_IDX = 9
F_CONT = 3
NC = 2
NS = 16
L = 16
NW = NC * NS
BPW = B // NW
NI = BPW * F_IDX
NCONT = BPW * F_CONT
GROUPS = BPW // L

_mesh = plsc.VectorSubcoreMesh(core_axis_name="c", subcore_axis_name="s")


@functools.partial(
    pl.kernel,
    mesh=_mesh,
    out_type=jax.ShapeDtypeStruct((B,), jnp.float32),
    scratch_types=[
        pltpu.VMEM((NI,), jnp.int32),
        pltpu.VMEM((NI,), jnp.float32),
        pltpu.VMEM((NCONT,), jnp.float32),
        pltpu.VMEM((BPW,), jnp.float32),
        pltpu.VMEM((L,), jnp.float32),
        pltpu.SemaphoreType.DMA,
    ],
)
def _fl_kernel(table_hbm, idx_hbm, cont_hbm, bias_hbm, out_hbm,
               idx_v, vals_v, cont_v, out_v, bias_v, sem):
    wid = lax.axis_index("s") * NC + lax.axis_index("c")
    pltpu.sync_copy(bias_hbm, bias_v)
    pltpu.sync_copy(idx_hbm.at[pl.ds(wid * NI, NI)], idx_v)
    pltpu.sync_copy(cont_hbm.at[pl.ds(wid * NCONT, NCONT)], cont_v)
    pltpu.async_copy(table_hbm.at[idx_v], vals_v, sem).wait()
    bv = bias_v[...]
    for g in range(GROUPS):
        o = g * L
        acc = bv
        for f in range(6):
            acc = acc + vals_v[pl.ds(f * BPW + o, L)]
        for k in range(F_CONT):
            acc = acc + (vals_v[pl.ds((6 + k) * BPW + o, L)]
                         * cont_v[pl.ds(k * BPW + o, L)])
        out_v[pl.ds(o, L)] = acc
    pltpu.sync_copy(out_v, out_hbm.at[pl.ds(wid * BPW, BPW)])


def kernel(x, fc_weight, bias):
    x32 = x.astype(jnp.int32)
    idx = x32[:, :F_IDX]
    cont = x32[:, F_IDX:].astype(jnp.float32)
    idx_arr = idx.reshape(NW, BPW, F_IDX).transpose(0, 2, 1).reshape(NW * NI)
    cont_arr = cont.reshape(NW, BPW, F_CONT).transpose(0, 2, 1).reshape(NW * NCONT)
    table = fc_weight.reshape(-1).astype(jnp.float32)
    bias_vec = jnp.broadcast_to(bias.astype(jnp.float32).reshape(-1), (L,))
    out = _fl_kernel(table, idx_arr, cont_arr, bias_vec)
    return out.reshape(B, 1)
